# Initial kernel scaffold; baseline (speedup 1.0000x reference)
import jax
import jax.numpy as jnp
from jax.experimental import pallas as pl


def kernel(token_ids, weight):
    raise NotImplementedError("write your pallas kernel here")


# speedup vs baseline: 1.5611x; 1.5611x over previous
import functools

import jax
import jax.numpy as jnp
from jax import lax
from jax.experimental import pallas as pl
from jax.experimental.pallas import tpu as pltpu
from jax.experimental.pallas import tpu_sc as plsc

NUM_EMBEDDINGS = 1000000
EMBEDDING_DIM = 32
BATCH = 16384
N_FIELDS = 26

NUM_LOOKUPS = BATCH * N_FIELDS
NC, NS = 2, 16
NW = NC * NS
B_PER_W = NUM_LOOKUPS // NW
N_CHUNKS = 8
CHUNK = B_PER_W // N_CHUNKS


def _embed_sc(idx_hbm, table_hbm, out_hbm, idx_v, rows_v, sem):
    wid = lax.axis_index("s") * NC + lax.axis_index("c")
    for c in range(N_CHUNKS):
        pltpu.sync_copy(idx_hbm.at[wid, c], idx_v)
        pltpu.async_copy(table_hbm.at[idx_v], rows_v, sem).wait()
        pltpu.sync_copy(rows_v, out_hbm.at[pl.ds(wid * B_PER_W + c * CHUNK, CHUNK)])


@jax.jit
def kernel(token_ids, weight):
    idx = token_ids.reshape(NW, N_CHUNKS, CHUNK).astype(jnp.int32)
    mesh = plsc.VectorSubcoreMesh(core_axis_name="c", subcore_axis_name="s")
    out = pl.kernel(
        _embed_sc,
        mesh=mesh,
        out_type=jax.ShapeDtypeStruct((NUM_LOOKUPS, EMBEDDING_DIM), jnp.float32),
        scratch_types=[
            pltpu.VMEM((CHUNK,), jnp.int32),
            pltpu.VMEM((CHUNK, EMBEDDING_DIM), jnp.float32),
            pltpu.SemaphoreType.DMA,
        ],
        compiler_params=pltpu.CompilerParams(use_tc_tiling_on_sc=False),
    )(idx, weight)
    return out.reshape(BATCH, N_FIELDS, EMBEDDING_DIM)


# speedup vs baseline: 1.5665x; 1.0035x over previous
import functools

import jax
import jax.numpy as jnp
from jax import lax
from jax.experimental import pallas as pl
from jax.experimental.pallas import tpu as pltpu
from jax.experimental.pallas import tpu_sc as plsc

NUM_EMBEDDINGS = 1000000
EMBEDDING_DIM = 32
BATCH = 16384
N_FIELDS = 26

NUM_LOOKUPS = BATCH * N_FIELDS
NC, NS = 2, 16
NW = NC * NS
B_PER_W = NUM_LOOKUPS // NW
N_CHUNKS = 8
CHUNK = B_PER_W // N_CHUNKS


def _embed_sc(idx_hbm, table_hbm, out_hbm, idx_v, rows0, rows1, gsem0, gsem1,
              wsem0, wsem1):
    wid = lax.axis_index("s") * NC + lax.axis_index("c")
    base = wid * B_PER_W
    rows = (rows0, rows1)
    gsem = (gsem0, gsem1)
    wsem = (wsem0, wsem1)

    pltpu.sync_copy(idx_hbm.at[wid], idx_v)

    def gather(c):
        return pltpu.make_async_copy(
            table_hbm.at[idx_v.at[pl.ds(c * CHUNK, CHUNK)]],
            rows[c % 2], gsem[c % 2])

    def writeback(c):
        return pltpu.make_async_copy(
            rows[c % 2], out_hbm.at[pl.ds(base + c * CHUNK, CHUNK)],
            wsem[c % 2])

    gather(0).start()
    for c in range(N_CHUNKS):
        gather(c).wait()
        if c + 1 < N_CHUNKS:
            if c >= 1:
                writeback(c - 1).wait()
            gather(c + 1).start()
        writeback(c).start()
    writeback(N_CHUNKS - 2).wait()
    writeback(N_CHUNKS - 1).wait()


@jax.jit
def kernel(token_ids, weight):
    idx = token_ids.reshape(NW, B_PER_W).astype(jnp.int32)
    mesh = plsc.VectorSubcoreMesh(core_axis_name="c", subcore_axis_name="s")
    out = pl.kernel(
        _embed_sc,
        mesh=mesh,
        out_type=jax.ShapeDtypeStruct((NUM_LOOKUPS, EMBEDDING_DIM), jnp.float32),
        scratch_types=[
            pltpu.VMEM((B_PER_W,), jnp.int32),
            pltpu.VMEM((CHUNK, EMBEDDING_DIM), jnp.float32),
            pltpu.VMEM((CHUNK, EMBEDDING_DIM), jnp.float32),
            pltpu.SemaphoreType.DMA,
            pltpu.SemaphoreType.DMA,
            pltpu.SemaphoreType.DMA,
            pltpu.SemaphoreType.DMA,
        ],
        compiler_params=pltpu.CompilerParams(use_tc_tiling_on_sc=False),
    )(idx, weight)
    return out.reshape(BATCH, N_FIELDS, EMBEDDING_DIM)
